# baseline (speedup 1.0000x reference)
import functools

import jax
import jax.numpy as jnp
from jax import lax
from jax.experimental import pallas as pl
from jax.experimental.pallas import tpu as pltpu
from jax.experimental.pallas import tpu_sc as plsc

B, T0, T1, D = 8, 2048, 128, 512
NW = 32
PW = (B * T0) // NW
CH = 32
NCH = PW // CH
XS = 2
US = 2
YS = 3
L = 16
NV = D // L

_mesh = plsc.VectorSubcoreMesh(core_axis_name="c", subcore_axis_name="s")


@functools.partial(
    pl.kernel,
    out_type=jax.ShapeDtypeStruct((B * T0, 2 * D), jnp.float32),
    mesh=_mesh,
    scratch_types=[
        pltpu.VMEM((T1,), jnp.int32),
        pltpu.VMEM((PW + L,), jnp.int32),
        pltpu.VMEM((XS, CH, D), jnp.float32),
        pltpu.VMEM((US * CH * D,), jnp.float32),
        pltpu.VMEM((YS, CH, D), jnp.float32),
    ] + [pltpu.SemaphoreType.DMA] * (2 * XS + US + YS),
    compiler_params=pltpu.CompilerParams(needs_layout_passes=False),
)
def _merge_sc(x0_hbm, pos_hbm, x1_hbm, out_hbm, pos_v, idx_v, xbuf, ubuf,
              ybuf, *sems):
    xis = sems[:XS]
    xos = sems[XS:2 * XS]
    uis = sems[2 * XS:2 * XS + US]
    yos = sems[2 * XS + US:]
    cid = lax.axis_index("c")
    sid = lax.axis_index("s")
    wid = sid * 2 + cid
    base = wid * PW
    b = base // T0
    t0 = base % T0

    def x_in(c, s):
        return pltpu.async_copy(
            x0_hbm.at[pl.ds(base + c * CH, CH)], xbuf.at[s], xis[s])

    def x_out(c, s):
        return pltpu.async_copy(
            xbuf.at[s],
            out_hbm.at[pl.ds(base + c * CH, CH), pl.ds(0, D)], xos[s])

    def y_out(c, s):
        return pltpu.async_copy(
            ybuf.at[s],
            out_hbm.at[pl.ds(base + c * CH, CH), pl.ds(D, D)], yos[s])

    xh_in = [None] * NCH
    xh_out = [None] * NCH
    for c in range(XS):
        xh_in[c] = x_in(c, c)
    pltpu.sync_copy(pos_hbm.at[pl.ds(b * T1, T1)], pos_v)

    lanes = lax.iota(jnp.int32, L)
    for v in range(PW // L):
        t_vec = t0 + v * L + lanes
        j = jnp.zeros((L,), jnp.int32)
        for step in (64, 32, 16, 8, 4, 2, 1):
            cand = j + step
            vals = plsc.load_gather(pos_v, [cand])
            j = jnp.where(vals <= t_vec, cand, j)
        idx_v[pl.ds(v * L, L)] = j

    lo = [None] * NCH
    for c in range(NCH):
        lo[c] = jnp.minimum(idx_v[pl.ds(c * CH, L)][0], T1 - CH)

    def u_in(c, s):
        return pltpu.async_copy(
            x1_hbm.at[pl.ds((b * T1 + lo[c]) * D, CH * D)],
            ubuf.at[pl.ds(s * CH * D, CH * D)], uis[s])

    uh = [None] * NCH
    yh = [None] * NCH
    for c in range(US):
        uh[c] = u_in(c, c)

    for c in range(NCH):
        s = c % XS
        xh_in[c].wait()
        xh_out[c] = x_out(c, s)
        if c + XS < NCH:
            xh_out[c].wait()
            xh_in[c + XS] = x_in(c + XS, s)
        us_ = c % US
        ys_ = c % YS
        uh[c].wait()
        if c >= YS:
            yh[c - YS].wait()
        ubase = us_ * CH * D - lo[c] * D

        def body(r, _):
            off = idx_v[pl.ds(c * CH + r, L)][0] * D + ubase
            for k in range(NV):
                ybuf[ys_, r, pl.ds(k * L, L)] = ubuf[pl.ds(off + k * L, L)]
            return 0

        lax.fori_loop(0, CH, body, 0)
        yh[c] = y_out(c, ys_)
        if c + US < NCH:
            uh[c + US] = u_in(c + US, us_)
    for c in range(NCH - YS, NCH):
        yh[c].wait()
    xh_out[NCH - 1].wait()


def kernel(x0, pos0, x1):
    x0f = jnp.reshape(x0, (B * T0, D))
    posf = jnp.reshape(pos0[:, :T1], (B * T1,))
    x1f = jnp.reshape(x1, (B * T1 * D,))
    out = _merge_sc(x0f, posf, x1f)
    return jnp.reshape(out, (B, T0, 2 * D))


# speedup vs baseline: 1.4986x; 1.4986x over previous
import functools

import jax
import jax.numpy as jnp
from jax import lax
from jax.experimental import pallas as pl
from jax.experimental.pallas import tpu as pltpu
from jax.experimental.pallas import tpu_sc as plsc

B, T0, T1, D = 8, 2048, 128, 512
NW = 32
PW = (B * T0) // NW
CH = 32
NCH = PW // CH
XS = 2
US = 2
YS = 3
L = 16
NV = D // L

_mesh = plsc.VectorSubcoreMesh(core_axis_name="c", subcore_axis_name="s")


@functools.partial(
    pl.kernel,
    out_type=jax.ShapeDtypeStruct((B * T0, 2 * D), jnp.float32),
    mesh=_mesh,
    scratch_types=[
        pltpu.VMEM((T1,), jnp.int32),
        pltpu.VMEM((PW + L,), jnp.int32),
        pltpu.VMEM((XS, CH, D), jnp.float32),
        pltpu.VMEM_SHARED((16, US * CH * D), jnp.float32),
        pltpu.VMEM((YS, CH, D), jnp.float32),
    ] + [pltpu.SemaphoreType.DMA] * (2 * XS + US + YS + 1),
    compiler_params=pltpu.CompilerParams(needs_layout_passes=False),
)
def _merge_sc(x0_hbm, pos_hbm, x1_hbm, out_hbm, pos_v, idx_v, xbuf, ubuf,
              ybuf, *sems):
    xis = sems[:XS]
    xos = sems[XS:2 * XS]
    uis = sems[2 * XS:2 * XS + US]
    yos = sems[2 * XS + US:2 * XS + US + YS]
    esem = sems[-1]
    cid = lax.axis_index("c")
    sid = lax.axis_index("s")
    wid = sid * 2 + cid
    base = wid * PW
    b = base // T0
    t0 = base % T0

    def x_in(c, s):
        return pltpu.async_copy(
            x0_hbm.at[pl.ds(base + c * CH, CH)], xbuf.at[s], xis[s])

    def x_out(c, s):
        return pltpu.async_copy(
            xbuf.at[s],
            out_hbm.at[pl.ds(base + c * CH, CH), pl.ds(0, D)], xos[s])

    def y_out(c, s):
        return pltpu.async_copy(
            ybuf.at[s],
            out_hbm.at[pl.ds(base + c * CH, CH), pl.ds(D, D)], yos[s])

    xh_in = [None] * NCH
    xh_out = [None] * NCH
    for c in range(XS):
        xh_in[c] = x_in(c, c)
    pltpu.sync_copy(pos_hbm.at[pl.ds(b * T1, T1)], pos_v)

    lanes = lax.iota(jnp.int32, L)
    for v in range(PW // L):
        t_vec = t0 + v * L + lanes
        j = jnp.zeros((L,), jnp.int32)
        for step in (64, 32, 16, 8, 4, 2, 1):
            cand = j + step
            vals = plsc.load_gather(pos_v, [cand])
            j = jnp.where(vals <= t_vec, cand, j)
        idx_v[pl.ds(v * L, L)] = j

    lo = [None] * NCH
    for c in range(NCH):
        lo[c] = jnp.minimum(idx_v[pl.ds(c * CH, L)][0], T1 - CH)

    def u_in(c, s):
        return pltpu.async_copy(
            x1_hbm.at[pl.ds((b * T1 + lo[c]) * D, CH * D)],
            ubuf.at[sid, pl.ds(s * CH * D, CH * D)], uis[s])

    uh = [None] * NCH
    yh = [None] * NCH
    for c in range(US):
        uh[c] = u_in(c, c)

    for c in range(NCH):
        s = c % XS
        xh_in[c].wait()
        xh_out[c] = x_out(c, s)
        if c + XS < NCH:
            xh_out[c].wait()
            xh_in[c + XS] = x_in(c + XS, s)
        us_ = c % US
        ys_ = c % YS
        uh[c].wait()
        if c >= YS:
            yh[c - YS].wait()
        ubase = us_ * CH * D - lo[c] * D

        def body(r, _):
            off = idx_v[pl.ds(c * CH + r, L)][0] * D + ubase
            pltpu.async_copy(ubuf.at[sid, pl.ds(off, D)], ybuf.at[ys_, r], esem)
            return 0

        lax.fori_loop(0, CH, body, 0)
        pltpu.make_async_copy(
            out_hbm.at[pl.ds(0, CH), pl.ds(0, D)],
            ybuf.at[ys_], esem).wait()
        yh[c] = y_out(c, ys_)
        if c + US < NCH:
            uh[c + US] = u_in(c + US, us_)
    for c in range(NCH - YS, NCH):
        yh[c].wait()
    xh_out[NCH - 1].wait()


def kernel(x0, pos0, x1):
    x0f = jnp.reshape(x0, (B * T0, D))
    posf = jnp.reshape(pos0[:, :T1], (B * T1,))
    x1f = jnp.reshape(x1, (B * T1 * D,))
    out = _merge_sc(x0f, posf, x1f)
    return jnp.reshape(out, (B, T0, 2 * D))


# speedup vs baseline: 1.6521x; 1.1024x over previous
import functools

import jax
import jax.numpy as jnp
from jax import lax
from jax.experimental import pallas as pl
from jax.experimental.pallas import tpu as pltpu
from jax.experimental.pallas import tpu_sc as plsc

B, T0, T1, D = 8, 2048, 128, 512
NW = 32
PW = (B * T0) // NW
CH = 32
NCH = PW // CH
XS = 2
US = 2
YS = 3
L = 16
NV = D // L

_mesh = plsc.VectorSubcoreMesh(core_axis_name="c", subcore_axis_name="s")


@functools.partial(
    pl.kernel,
    out_type=jax.ShapeDtypeStruct((B * T0, 2 * D), jnp.float32),
    mesh=_mesh,
    scratch_types=[
        pltpu.VMEM((T1,), jnp.int32),
        pltpu.VMEM((PW + L,), jnp.int32),
        pltpu.VMEM((XS, CH, D), jnp.float32),
        pltpu.VMEM_SHARED((16, US * CH * D), jnp.float32),
        pltpu.VMEM((YS, CH, D), jnp.float32),
    ] + [pltpu.SemaphoreType.DMA] * (2 * XS + US + YS + 1),
    compiler_params=pltpu.CompilerParams(needs_layout_passes=False),
)
def _merge_sc(x0_hbm, pos_hbm, x1_hbm, out_hbm, pos_v, idx_v, xbuf, ubuf,
              ybuf, *sems):
    xis = sems[:XS]
    xos = sems[XS:2 * XS]
    uis = sems[2 * XS:2 * XS + US]
    yos = sems[2 * XS + US:2 * XS + US + YS]
    esem = sems[-1]
    cid = lax.axis_index("c")
    sid = lax.axis_index("s")
    wid = sid * 2 + cid
    base = wid * PW
    b = base // T0
    t0 = base % T0

    def x_in(c, s):
        return pltpu.async_copy(
            x0_hbm.at[pl.ds(base + c * CH, CH)], xbuf.at[s], xis[s])

    def x_out(c, s):
        return pltpu.async_copy(
            xbuf.at[s],
            out_hbm.at[pl.ds(base + c * CH, CH), pl.ds(0, D)], xos[s])

    def y_out(c, s):
        return pltpu.async_copy(
            ybuf.at[s],
            out_hbm.at[pl.ds(base + c * CH, CH), pl.ds(D, D)], yos[s])

    xh_in = [None] * NCH
    xh_out = [None] * NCH
    for c in range(XS):
        xh_in[c] = x_in(c, c)
    pltpu.sync_copy(pos_hbm.at[pl.ds(b * T1, T1)], pos_v)

    lanes = lax.iota(jnp.int32, L)
    for v in range(PW // L):
        t_vec = t0 + v * L + lanes
        j = jnp.zeros((L,), jnp.int32)
        for step in (64, 32, 16, 8, 4, 2, 1):
            cand = j + step
            vals = plsc.load_gather(pos_v, [cand])
            j = jnp.where(vals <= t_vec, cand, j)
        idx_v[pl.ds(v * L, L)] = j

    P8 = 8
    lo = [None] * NCH
    need = [None] * NCH
    for c in range(NCH):
        lo[c] = jnp.minimum(idx_v[pl.ds(c * CH, L)][0], T1 - CH)
        hi = idx_v[pl.ds(c * CH + CH - L, L)][L - 1]
        need[c] = hi - lo[c] + 1

    def _u_part(c, s, p):
        src = x1_hbm.at[pl.ds((b * T1 + lo[c] + p * P8) * D, P8 * D)]
        dst = ubuf.at[sid, pl.ds(s * CH * D + p * P8 * D, P8 * D)]
        return src, dst

    def u_in(c, s):
        for p in range(1, CH // P8):
            @pl.when(need[c] > p * P8)
            def _():
                src, dst = _u_part(c, s, p)
                pltpu.async_copy(src, dst, uis[s])
        src, dst = _u_part(c, s, 0)
        return pltpu.async_copy(src, dst, uis[s])

    def u_wait(c, s, h):
        h.wait()
        for p in range(1, CH // P8):
            @pl.when(need[c] > p * P8)
            def _():
                src, dst = _u_part(c, s, p)
                pltpu.make_async_copy(src, dst, uis[s]).wait()

    uh = [None] * NCH
    yh = [None] * NCH
    for c in range(US):
        uh[c] = u_in(c, c)

    for c in range(NCH):
        s = c % XS
        xh_in[c].wait()
        xh_out[c] = x_out(c, s)
        if c + XS < NCH:
            xh_out[c].wait()
            xh_in[c + XS] = x_in(c + XS, s)
        us_ = c % US
        ys_ = c % YS
        u_wait(c, us_, uh[c])
        if c >= YS:
            yh[c - YS].wait()
        ubase = us_ * CH * D - lo[c] * D

        def body(r, _):
            off = idx_v[pl.ds(c * CH + r, L)][0] * D + ubase
            pltpu.async_copy(ubuf.at[sid, pl.ds(off, D)], ybuf.at[ys_, r], esem)
            return 0

        lax.fori_loop(0, CH, body, 0)
        pltpu.make_async_copy(
            out_hbm.at[pl.ds(0, CH), pl.ds(0, D)],
            ybuf.at[ys_], esem).wait()
        yh[c] = y_out(c, ys_)
        if c + US < NCH:
            uh[c + US] = u_in(c + US, us_)
    for c in range(NCH - YS, NCH):
        yh[c].wait()
    xh_out[NCH - 1].wait()


def kernel(x0, pos0, x1):
    x0f = jnp.reshape(x0, (B * T0, D))
    posf = jnp.reshape(pos0[:, :T1], (B * T1,))
    x1f = jnp.reshape(x1, (B * T1 * D,))
    out = _merge_sc(x0f, posf, x1f)
    return jnp.reshape(out, (B, T0, 2 * D))


# speedup vs baseline: 1.6750x; 1.0139x over previous
import functools

import jax
import jax.numpy as jnp
from jax import lax
from jax.experimental import pallas as pl
from jax.experimental.pallas import tpu as pltpu
from jax.experimental.pallas import tpu_sc as plsc

B, T0, T1, D = 8, 2048, 128, 512
NW = 32
PW = (B * T0) // NW
CH = 32
NCH = PW // CH
XS = 3
US = 2
YS = 2
L = 16
NV = D // L

_mesh = plsc.VectorSubcoreMesh(core_axis_name="c", subcore_axis_name="s")


@functools.partial(
    pl.kernel,
    out_type=jax.ShapeDtypeStruct((B * T0, 2 * D), jnp.float32),
    mesh=_mesh,
    scratch_types=[
        pltpu.VMEM((T1,), jnp.int32),
        pltpu.VMEM((PW + L,), jnp.int32),
        pltpu.VMEM((XS, CH, D), jnp.float32),
        pltpu.VMEM_SHARED((16, US * CH * D), jnp.float32),
        pltpu.VMEM((YS, CH, D), jnp.float32),
    ] + [pltpu.SemaphoreType.DMA] * (2 * XS + US + YS + 1),
    compiler_params=pltpu.CompilerParams(needs_layout_passes=False),
)
def _merge_sc(x0_hbm, pos_hbm, x1_hbm, out_hbm, pos_v, idx_v, xbuf, ubuf,
              ybuf, *sems):
    xis = sems[:XS]
    xos = sems[XS:2 * XS]
    uis = sems[2 * XS:2 * XS + US]
    yos = sems[2 * XS + US:2 * XS + US + YS]
    esem = sems[-1]
    cid = lax.axis_index("c")
    sid = lax.axis_index("s")
    wid = sid * 2 + cid
    base = wid * PW
    b = base // T0
    t0 = base % T0

    def x_in(c, s):
        return pltpu.async_copy(
            x0_hbm.at[pl.ds(base + c * CH, CH)], xbuf.at[s], xis[s])

    def x_out(c, s):
        return pltpu.async_copy(
            xbuf.at[s],
            out_hbm.at[pl.ds(base + c * CH, CH), pl.ds(0, D)], xos[s])

    def y_out(c, s):
        return pltpu.async_copy(
            ybuf.at[s],
            out_hbm.at[pl.ds(base + c * CH, CH), pl.ds(D, D)], yos[s])

    xh_in = [None] * NCH
    xh_out = [None] * NCH
    for c in range(XS):
        xh_in[c] = x_in(c, c)
    pltpu.sync_copy(pos_hbm.at[pl.ds(b * T1, T1)], pos_v)

    lanes = lax.iota(jnp.int32, L)
    for v in range(PW // L):
        t_vec = t0 + v * L + lanes
        j = jnp.zeros((L,), jnp.int32)
        for step in (64, 32, 16, 8, 4, 2, 1):
            cand = j + step
            vals = plsc.load_gather(pos_v, [cand])
            j = jnp.where(vals <= t_vec, cand, j)
        idx_v[pl.ds(v * L, L)] = j

    P8 = 8
    lo = [None] * NCH
    need = [None] * NCH
    for c in range(NCH):
        lo[c] = jnp.minimum(idx_v[pl.ds(c * CH, L)][0], T1 - CH)
        hi = idx_v[pl.ds(c * CH + CH - L, L)][L - 1]
        need[c] = hi - lo[c] + 1

    def _u_part(c, s, p):
        src = x1_hbm.at[pl.ds((b * T1 + lo[c] + p * P8) * D, P8 * D)]
        dst = ubuf.at[sid, pl.ds(s * CH * D + p * P8 * D, P8 * D)]
        return src, dst

    def u_in(c, s):
        for p in range(1, CH // P8):
            @pl.when(need[c] > p * P8)
            def _():
                src, dst = _u_part(c, s, p)
                pltpu.async_copy(src, dst, uis[s])
        src, dst = _u_part(c, s, 0)
        return pltpu.async_copy(src, dst, uis[s])

    def u_wait(c, s, h):
        h.wait()
        for p in range(1, CH // P8):
            @pl.when(need[c] > p * P8)
            def _():
                src, dst = _u_part(c, s, p)
                pltpu.make_async_copy(src, dst, uis[s]).wait()

    uh = [None] * NCH
    yh = [None] * NCH
    for c in range(US):
        uh[c] = u_in(c, c)

    for c in range(NCH):
        s = c % XS
        xh_in[c].wait()
        xh_out[c] = x_out(c, s)
        if c + XS < NCH:
            xh_out[c].wait()
            xh_in[c + XS] = x_in(c + XS, s)
        us_ = c % US
        ys_ = c % YS
        u_wait(c, us_, uh[c])
        if c >= YS:
            yh[c - YS].wait()
        ubase = us_ * CH * D - lo[c] * D

        def body(r, _):
            off = idx_v[pl.ds(c * CH + r, L)][0] * D + ubase
            pltpu.async_copy(ubuf.at[sid, pl.ds(off, D)], ybuf.at[ys_, r], esem)
            return 0

        lax.fori_loop(0, CH, body, 0)
        pltpu.make_async_copy(
            out_hbm.at[pl.ds(0, CH), pl.ds(0, D)],
            ybuf.at[ys_], esem).wait()
        yh[c] = y_out(c, ys_)
        if c + US < NCH:
            uh[c + US] = u_in(c + US, us_)
    for c in range(NCH - YS, NCH):
        yh[c].wait()
    xh_out[NCH - 1].wait()


def kernel(x0, pos0, x1):
    x0f = jnp.reshape(x0, (B * T0, D))
    posf = jnp.reshape(pos0[:, :T1], (B * T1,))
    x1f = jnp.reshape(x1, (B * T1 * D,))
    out = _merge_sc(x0f, posf, x1f)
    return jnp.reshape(out, (B, T0, 2 * D))
